# baseline (speedup 1.0000x reference)
import functools

import jax
import jax.numpy as jnp
from jax import lax
from jax.experimental import pallas as pl
from jax.experimental.pallas import tpu as pltpu
from jax.experimental.pallas import tpu_sc as plsc

IN_DIM = 1000000
DEPTH = 64
SEQ = 200
BATCH = 4096
ROWS = BATCH * SEQ
NC = 2
NS = 16
LANES = 16
NW = NC * NS
PER_W = ROWS // NW
CHUNK = 400
NCHUNK = PER_W // CHUNK
GSZ = 80
NG = CHUNK // GSZ
NGRP = CHUNK // LANES
TOTAL_CHUNKS = ROWS // CHUNK
VCHUNK = CHUNK * DEPTH // 128


def _pos_encoding():
    half = DEPTH // 2
    positions = jnp.arange(SEQ, dtype=jnp.float32)[:, None]
    depths = jnp.arange(half, dtype=jnp.float32)[None, :] / half
    angle_rates = 1.0 / 10000.0 ** depths
    angle_rads = positions * angle_rates
    return jnp.concatenate([jnp.sin(angle_rads), jnp.cos(angle_rads)], axis=-1)


def _make_sc_kernel():
    mesh = plsc.VectorSubcoreMesh(core_axis_name="c", subcore_axis_name="s")

    @functools.partial(
        pl.kernel,
        mesh=mesh,
        compiler_params=pltpu.CompilerParams(needs_layout_passes=False),
        out_type=jax.ShapeDtypeStruct((TOTAL_CHUNKS, VCHUNK, 128), jnp.float32),
        scratch_types=[
            pltpu.VMEM((NG, GSZ), jnp.int32),
            pltpu.VMEM((NG, GSZ), jnp.int32),
            pltpu.VMEM((CHUNK, 128), jnp.float32),
            pltpu.VMEM((VCHUNK, 128), jnp.float32),
            pltpu.VMEM((DEPTH, CHUNK), jnp.float32),
            pltpu.SemaphoreType.DMA,
        ],
    )
    def k(idx_hbm, table_hbm, pos_hbm, out_hbm, idxr_v, idx2_v, rows2_v,
          out_v, pos_v, sem):
        wid = lax.axis_index("s") * NC + lax.axis_index("c")
        pltpu.sync_copy(pos_hbm, pos_v)
        iota16 = lax.iota(jnp.int32, LANES)

        def chunk_body(c, carry):
            cg = wid * NCHUNK + c
            pltpu.sync_copy(idx_hbm.at[cg], idxr_v)
            for g in range(NG):
                for o in range(0, GSZ, LANES):
                    sl = pl.ds(o, LANES)
                    idx2_v[g, sl] = lax.shift_right_logical(idxr_v[g, sl], 1)
            copies = [
                pltpu.async_copy(
                    table_hbm.at[idx2_v.at[g]],
                    rows2_v.at[pl.ds(g * GSZ, GSZ)],
                    sem,
                )
                for g in range(NG)
            ]
            for cp in copies:
                cp.wait()

            def grp_body(m, gcarry):
                i0 = m * LANES
                g = i0 // GSZ
                o = i0 % GSZ
                raw16 = idxr_v[g, pl.ds(o, LANES)]
                par16 = lax.bitwise_and(raw16, 1)
                k16 = i0 + iota16
                srow16 = k16
                scol0 = par16 * DEPTH
                orow16 = lax.shift_right_logical(k16, 1)
                ocol0 = lax.bitwise_and(k16, 1) * DEPTH
                for d in range(DEPTH):
                    v = plsc.load_gather(rows2_v, [srow16, scol0 + d])
                    p = pos_v[d, pl.ds(i0, LANES)]
                    plsc.store_scatter(out_v, [orow16, ocol0 + d], v + p)
                return gcarry

            lax.fori_loop(0, NGRP, grp_body, 0)
            pltpu.sync_copy(out_v, out_hbm.at[cg])
            return carry

        lax.fori_loop(0, NCHUNK, chunk_body, 0)

    return k


def kernel(seq, table):
    idx = seq.astype(jnp.int32).reshape(TOTAL_CHUNKS, NG, GSZ)
    table2 = table.reshape(IN_DIM // 2, 128)
    pos_t = jnp.tile(_pos_encoding(), (CHUNK // SEQ, 1)).T
    out = _make_sc_kernel()(idx, table2, pos_t)
    return out.reshape(BATCH, SEQ, DEPTH)


# speedup vs baseline: 2.0408x; 2.0408x over previous
import functools

import jax
import jax.numpy as jnp
from jax import lax
from jax.experimental import pallas as pl
from jax.experimental.pallas import tpu as pltpu
from jax.experimental.pallas import tpu_sc as plsc

IN_DIM = 1000000
DEPTH = 64
SEQ = 200
BATCH = 4096
ROWS = BATCH * SEQ
NC = 2
NS = 16
LANES = 16
NW = NC * NS
PER_W = ROWS // NW
CHUNK = 400
NCHUNK = PER_W // CHUNK
GSZ = 80
NG = CHUNK // GSZ
TOTAL_CHUNKS = ROWS // CHUNK


def _pos_encoding():
    half = DEPTH // 2
    positions = jnp.arange(SEQ, dtype=jnp.float32)[:, None]
    depths = jnp.arange(half, dtype=jnp.float32)[None, :] / half
    angle_rates = 1.0 / 10000.0 ** depths
    angle_rads = positions * angle_rates
    return jnp.concatenate([jnp.sin(angle_rads), jnp.cos(angle_rads)], axis=-1)


def _make_sc_kernel():
    mesh = plsc.VectorSubcoreMesh(core_axis_name="c", subcore_axis_name="s")

    @functools.partial(
        pl.kernel,
        mesh=mesh,
        out_type=jax.ShapeDtypeStruct((TOTAL_CHUNKS, CHUNK, DEPTH), jnp.float32),
        scratch_types=[
            pltpu.VMEM((NG, GSZ), jnp.int32),
            pltpu.VMEM((NG, GSZ), jnp.int32),
            pltpu.VMEM((CHUNK, 128), jnp.float32),
            pltpu.VMEM((CHUNK, DEPTH), jnp.float32),
            pltpu.VMEM((SEQ, DEPTH), jnp.float32),
            pltpu.SemaphoreType.DMA,
        ],
    )
    def k(idx_hbm, table_hbm, pos_hbm, out_hbm, idxr_v, idx2_v, rows2_v,
          out_v, pos_v, sem):
        wid = lax.axis_index("s") * NC + lax.axis_index("c")
        pltpu.sync_copy(pos_hbm, pos_v)

        def chunk_body(c, carry):
            cg = wid * NCHUNK + c
            pltpu.sync_copy(idx_hbm.at[cg], idxr_v)
            for g in range(NG):
                for o in range(0, GSZ, LANES):
                    sl = pl.ds(o, LANES)
                    idx2_v[g, sl] = lax.shift_right_logical(idxr_v[g, sl], 1)
            copies = [
                pltpu.async_copy(
                    table_hbm.at[idx2_v.at[g]],
                    rows2_v.at[pl.ds(g * GSZ, GSZ)],
                    sem,
                )
                for g in range(NG)
            ]
            for cp in copies:
                cp.wait()

            for g in range(NG):
                p0 = (g * GSZ) % SEQ

                def grp_body(m, gcarry, g=g, p0=p0):
                    o0 = m * LANES
                    off16 = lax.bitwise_and(
                        idxr_v[g, pl.ds(o0, LANES)], 1) * DEPTH
                    for r in range(LANES):
                        off = off16[r]
                        k_row = g * GSZ + o0 + r
                        p = p0 + o0 + r
                        p = lax.select(p >= SEQ, p - SEQ, p)
                        for j in range(DEPTH // LANES):
                            out_v[k_row, pl.ds(j * LANES, LANES)] = (
                                rows2_v[k_row, pl.ds(off + j * LANES, LANES)]
                                + pos_v[p, pl.ds(j * LANES, LANES)]
                            )
                    return gcarry

                lax.fori_loop(0, GSZ // LANES, grp_body, 0)

            pltpu.sync_copy(out_v, out_hbm.at[cg])
            return carry

        lax.fori_loop(0, NCHUNK, chunk_body, 0)

    return k


def kernel(seq, table):
    idx = seq.astype(jnp.int32).reshape(TOTAL_CHUNKS, NG, GSZ)
    table2 = table.reshape(IN_DIM // 2, 128)
    pos = _pos_encoding()
    out = _make_sc_kernel()(idx, table2, pos)
    return out.reshape(BATCH, SEQ, DEPTH)
